# Initial kernel scaffold; baseline (speedup 1.0000x reference)
import jax
import jax.numpy as jnp
from jax.experimental import pallas as pl


def kernel(feat, edge_index, W1, b1, W2, b2):
    raise NotImplementedError("write your pallas kernel here")


# speedup vs baseline: 3.6730x; 3.6730x over previous
import functools

import jax
import jax.numpy as jnp
from jax import lax
from jax.experimental import pallas as pl
from jax.experimental.pallas import tpu as pltpu
from jax.experimental.pallas import tpu_sc as plsc

_K = 80
_MESH = plsc.VectorSubcoreMesh(core_axis_name="c", subcore_axis_name="s")
_RCH = 200
_BM = 1000


def _deg_call(src, dst, o2, z128, n):
    e = src.shape[0]
    cpt = e // (32 * _K)
    nchunk = n // _RCH
    tmax = (nchunk + 15) // 16

    @functools.partial(
        pl.kernel,
        out_type=jax.ShapeDtypeStruct((2 * n, 128), jnp.float32),
        mesh=_MESH,
        scratch_types=[
            pltpu.VMEM((_K,), jnp.int32),
            pltpu.VMEM((_K,), jnp.int32),
            pltpu.VMEM((_K, 128), jnp.float32),
            pltpu.VMEM((_K, 128), jnp.float32),
            pltpu.VMEM((_RCH, 128), jnp.float32),
            pltpu.VMEM_SHARED((n, 128), jnp.float32),
        ],
    )
    def deg_kernel(src_hbm, dst_hbm, o_hbm, z_hbm, out_hbm,
                   idxs, idxd, ones_s, ones_d, stage, acc):
        c = lax.axis_index("c")
        s = lax.axis_index("s")

        pltpu.sync_copy(z_hbm, stage)
        pltpu.sync_copy(o_hbm.at[pl.ds(0, _K)], ones_s)
        pltpu.sync_copy(o_hbm.at[pl.ds(_K, _K)], ones_d)

        @pl.loop(0, tmax)
        def _(t):
            q = s + t * 16

            @pl.when(q < nchunk)
            def _():
                pltpu.sync_copy(stage, acc.at[pl.ds(q * _RCH, _RCH)])

        plsc.subcore_barrier()

        w = c * 16 + s
        @pl.loop(0, cpt)
        def _(j):
            off = (w * cpt + j) * _K
            pltpu.sync_copy(src_hbm.at[pl.ds(off, _K)], idxs)
            pltpu.sync_copy(ones_s, acc.at[idxs], add=True)
            pltpu.sync_copy(dst_hbm.at[pl.ds(off, _K)], idxd)
            pltpu.sync_copy(ones_d, acc.at[idxd], add=True)

        plsc.subcore_barrier()

        @pl.loop(0, tmax)
        def _(t):
            q = s + t * 16

            @pl.when(q < nchunk)
            def _():
                r0 = q * _RCH
                pltpu.sync_copy(acc.at[pl.ds(r0, _RCH)], stage)
                pltpu.sync_copy(stage, out_hbm.at[pl.ds(c * n + r0, _RCH)])

    return deg_kernel(src, dst, o2, z128)


def _edge_call(src, dst, tbl_a, tbl_b, z128, n, layer1):
    e = src.shape[0]
    cpt = e // ((16 if layer1 else 32) * _K)
    nchunk = n // _RCH
    tmax = (nchunk + 15) // 16

    @functools.partial(
        pl.kernel,
        out_type=jax.ShapeDtypeStruct((2 * n, 128), jnp.float32),
        mesh=_MESH,
        scratch_types=[
            pltpu.VMEM((_K,), jnp.int32),
            pltpu.VMEM((_K,), jnp.int32),
            pltpu.VMEM((_K, 128), jnp.float32),
            pltpu.VMEM((_RCH, 128), jnp.float32),
            pltpu.VMEM_SHARED((n, 128), jnp.float32),
            pltpu.SemaphoreType.DMA,
        ],
    )
    def edge_kernel(src_hbm, dst_hbm, ta_hbm, tb_hbm, z_hbm, out_hbm,
                    idxs, idxd, rows, stage, acc, sem):
        c = lax.axis_index("c")
        s = lax.axis_index("s")

        pltpu.sync_copy(z_hbm, stage)

        @pl.loop(0, tmax)
        def _(t):
            q = s + t * 16

            @pl.when(q < nchunk)
            def _():
                pltpu.sync_copy(stage, acc.at[pl.ds(q * _RCH, _RCH)])

        plsc.subcore_barrier()

        base = (s if layer1 else c * 16 + s) * cpt

        @pl.loop(0, cpt)
        def _(j):
            off = (base + j) * _K
            pltpu.sync_copy(src_hbm.at[pl.ds(off, _K)], idxs)
            pltpu.sync_copy(dst_hbm.at[pl.ds(off, _K)], idxd)

            @pl.when(c == 0)
            def _():
                pltpu.async_copy(ta_hbm.at[idxs], rows, sem).wait()

            @pl.when(c == 1)
            def _():
                pltpu.async_copy(tb_hbm.at[idxs], rows, sem).wait()

            pltpu.sync_copy(rows, acc.at[idxd], add=True)

        plsc.subcore_barrier()

        @pl.loop(0, tmax)
        def _(t):
            q = s + t * 16

            @pl.when(q < nchunk)
            def _():
                r0 = q * _RCH
                pltpu.sync_copy(acc.at[pl.ds(r0, _RCH)], stage)
                pltpu.sync_copy(stage, out_hbm.at[pl.ds(c * n + r0, _RCH)])

    return edge_kernel(src, dst, tbl_a, tbl_b, z128)


def _norm(d0, d1, col):
    d = d0[:, col:col + 1] + d1[:, col:col + 1]
    return lax.rsqrt(jnp.maximum(d, 1.0))


def _mm1_call(feat, w1, dega, n):
    gm = n // _BM

    def body(x_ref, w_ref, d0, d1, o_ref):
        ns = _norm(d0[...], d1[...], 0)
        o_ref[...] = jnp.dot(x_ref[...] * ns, w_ref[...],
                             preferred_element_type=jnp.float32)

    return pl.pallas_call(
        body,
        grid=(gm, 2),
        in_specs=[
            pl.BlockSpec((_BM, 128), lambda i, j: (i, 0)),
            pl.BlockSpec((128, 128), lambda i, j: (0, j)),
            pl.BlockSpec((_BM, 128), lambda i, j: (i, 0)),
            pl.BlockSpec((_BM, 128), lambda i, j: (gm + i, 0)),
        ],
        out_specs=pl.BlockSpec((_BM, 128), lambda i, j: (j * gm + i, 0)),
        out_shape=jax.ShapeDtypeStruct((2 * n, 128), jnp.float32),
    )(feat, w1, dega, dega)


def _mm2_call(agg1, w2, b1r, dega, n):
    gm = n // _BM

    def body(a_ref, w_ref, b_ref, d0, d1, o_ref):
        k = pl.program_id(1)
        nd = _norm(d0[...], d1[...], 1)
        ns = _norm(d0[...], d1[...], 0)
        x = jnp.maximum(a_ref[...] * nd + b_ref[0], 0.0) * ns
        p = jnp.dot(x, w_ref[...], preferred_element_type=jnp.float32)

        @pl.when(k == 0)
        def _():
            o_ref[...] = p

        @pl.when(k > 0)
        def _():
            o_ref[...] += p

    return pl.pallas_call(
        body,
        grid=(gm, 2),
        in_specs=[
            pl.BlockSpec((_BM, 128), lambda i, k: (k * gm + i, 0)),
            pl.BlockSpec((128, 128), lambda i, k: (k, 0)),
            pl.BlockSpec((1, 1, 128), lambda i, k: (k, 0, 0)),
            pl.BlockSpec((_BM, 128), lambda i, k: (i, 0)),
            pl.BlockSpec((_BM, 128), lambda i, k: (gm + i, 0)),
        ],
        out_specs=pl.BlockSpec((_BM, 128), lambda i, k: (i, 0)),
        out_shape=jax.ShapeDtypeStruct((n, 128), jnp.float32),
    )(agg1, w2, b1r, dega, dega)


def _fin_call(agg2, b2r, dega, n):
    gm = n // _BM

    def body(a0_ref, a1_ref, d0, d1, b_ref, o_ref):
        nd = _norm(d0[...], d1[...], 1)
        o_ref[...] = jnp.maximum(
            (a0_ref[...] + a1_ref[...]) * nd + b_ref[...], 0.0)

    return pl.pallas_call(
        body,
        grid=(gm,),
        in_specs=[
            pl.BlockSpec((_BM, 128), lambda i: (i, 0)),
            pl.BlockSpec((_BM, 128), lambda i: (gm + i, 0)),
            pl.BlockSpec((_BM, 128), lambda i: (i, 0)),
            pl.BlockSpec((_BM, 128), lambda i: (gm + i, 0)),
            pl.BlockSpec((1, 128), lambda i: (0, 0)),
        ],
        out_specs=pl.BlockSpec((_BM, 128), lambda i: (i, 0)),
        out_shape=jax.ShapeDtypeStruct((n, 128), jnp.float32),
    )(agg2, agg2, dega, dega, b2r)


def kernel(feat, edge_index, W1, b1, W2, b2):
    n, in_dim = feat.shape
    e = edge_index.shape[1]
    hid = W1.shape[1]
    out_dim = W2.shape[1]
    assert (in_dim, hid, out_dim) == (128, 256, 128)
    assert e % (32 * _K) == 0 and n % _BM == 0 and n % (2 * _RCH) == 0

    src = edge_index[0]
    dst = edge_index[1]
    z128 = jnp.zeros((_RCH, 128), jnp.float32)
    o2 = jnp.zeros((2 * _K, 128), jnp.float32)
    o2 = o2.at[:_K, 0].set(1.0).at[_K:, 1].set(1.0)

    dega = _deg_call(src, dst, o2, z128, n)
    h1 = _mm1_call(feat, W1, dega, n)
    agg1 = _edge_call(src, dst, h1[:n], h1[n:], z128, n, True)
    h2 = _mm2_call(agg1, W2, b1.reshape(2, 1, 128), dega, n)
    agg2 = _edge_call(src, dst, h2, h2, z128, n, False)
    return _fin_call(agg2, b2.reshape(1, 128), dega, n)


# speedup vs baseline: 4.2073x; 1.1455x over previous
import functools

import jax
import jax.numpy as jnp
from jax import lax
from jax.experimental import pallas as pl
from jax.experimental.pallas import tpu as pltpu
from jax.experimental.pallas import tpu_sc as plsc

_K = 80
_MESH = plsc.VectorSubcoreMesh(core_axis_name="c", subcore_axis_name="s")
_RCH = 200
_BM = 1000


def _deg_call(src, dst, o2, z128, n):
    e = src.shape[0]
    cpt = e // (32 * _K)
    nchunk = n // _RCH
    tmax = (nchunk + 15) // 16

    @functools.partial(
        pl.kernel,
        out_type=jax.ShapeDtypeStruct((2 * n, 128), jnp.float32),
        mesh=_MESH,
        scratch_types=[
            pltpu.VMEM((_K,), jnp.int32),
            pltpu.VMEM((_K,), jnp.int32),
            pltpu.VMEM((_K, 128), jnp.float32),
            pltpu.VMEM((_K, 128), jnp.float32),
            pltpu.VMEM((_RCH, 128), jnp.float32),
            pltpu.VMEM_SHARED((n, 128), jnp.float32),
        ],
    )
    def deg_kernel(src_hbm, dst_hbm, o_hbm, z_hbm, out_hbm,
                   idxs, idxd, ones_s, ones_d, stage, acc):
        c = lax.axis_index("c")
        s = lax.axis_index("s")

        pltpu.sync_copy(z_hbm, stage)
        pltpu.sync_copy(o_hbm.at[pl.ds(0, _K)], ones_s)
        pltpu.sync_copy(o_hbm.at[pl.ds(_K, _K)], ones_d)

        @pl.loop(0, tmax)
        def _(t):
            q = s + t * 16

            @pl.when(q < nchunk)
            def _():
                pltpu.sync_copy(stage, acc.at[pl.ds(q * _RCH, _RCH)])

        plsc.subcore_barrier()

        w = c * 16 + s
        @pl.loop(0, cpt)
        def _(j):
            off = (w * cpt + j) * _K
            pltpu.sync_copy(src_hbm.at[pl.ds(off, _K)], idxs)
            pltpu.sync_copy(ones_s, acc.at[idxs], add=True)
            pltpu.sync_copy(dst_hbm.at[pl.ds(off, _K)], idxd)
            pltpu.sync_copy(ones_d, acc.at[idxd], add=True)

        plsc.subcore_barrier()

        @pl.loop(0, tmax)
        def _(t):
            q = s + t * 16

            @pl.when(q < nchunk)
            def _():
                r0 = q * _RCH
                pltpu.sync_copy(acc.at[pl.ds(r0, _RCH)], stage)
                pltpu.sync_copy(stage, out_hbm.at[pl.ds(c * n + r0, _RCH)])

    return deg_kernel(src, dst, o2, z128)


def _edge_call(src, dst, tbl_a, tbl_b, z128, n, layer1):
    e = src.shape[0]
    cpt = e // ((16 if layer1 else 32) * _K)
    nchunk = n // _RCH
    tmax = (nchunk + 15) // 16

    @functools.partial(
        pl.kernel,
        out_type=jax.ShapeDtypeStruct((2 * n, 128), jnp.float32),
        mesh=_MESH,
        scratch_types=[
            pltpu.VMEM((_K,), jnp.int32),
            pltpu.VMEM((_K,), jnp.int32),
            pltpu.VMEM((_K, 128), jnp.float32),
            pltpu.VMEM((_RCH, 128), jnp.float32),
            pltpu.VMEM_SHARED((n, 128), jnp.float32),
            pltpu.SemaphoreType.DMA,
        ],
    )
    def edge_kernel(src_hbm, dst_hbm, ta_hbm, tb_hbm, z_hbm, out_hbm,
                    idxs, idxd, rows, stage, acc, sem):
        c = lax.axis_index("c")
        s = lax.axis_index("s")

        pltpu.sync_copy(z_hbm, stage)

        @pl.loop(0, tmax)
        def _(t):
            q = s + t * 16

            @pl.when(q < nchunk)
            def _():
                pltpu.sync_copy(stage, acc.at[pl.ds(q * _RCH, _RCH)])

        plsc.subcore_barrier()

        base = (s if layer1 else c * 16 + s) * cpt

        @pl.loop(0, cpt)
        def _(j):
            off = (base + j) * _K
            pltpu.sync_copy(src_hbm.at[pl.ds(off, _K)], idxs)

            @pl.when(c == 0)
            def _():
                g = pltpu.async_copy(ta_hbm.at[idxs], rows, sem)
                pltpu.sync_copy(dst_hbm.at[pl.ds(off, _K)], idxd)
                g.wait()

            @pl.when(c == 1)
            def _():
                g = pltpu.async_copy(tb_hbm.at[idxs], rows, sem)
                pltpu.sync_copy(dst_hbm.at[pl.ds(off, _K)], idxd)
                g.wait()

            pltpu.sync_copy(rows, acc.at[idxd], add=True)

        plsc.subcore_barrier()

        @pl.loop(0, tmax)
        def _(t):
            q = s + t * 16

            @pl.when(q < nchunk)
            def _():
                r0 = q * _RCH
                pltpu.sync_copy(acc.at[pl.ds(r0, _RCH)], stage)
                pltpu.sync_copy(stage, out_hbm.at[pl.ds(c * n + r0, _RCH)])

    return edge_kernel(src, dst, tbl_a, tbl_b, z128)


def _norm(d0, d1, col):
    d = d0[:, col:col + 1] + d1[:, col:col + 1]
    return lax.rsqrt(jnp.maximum(d, 1.0))


def _mm1_call(feat, w1, dega, n):
    gm = n // _BM

    def body(x_ref, w_ref, d0, d1, o_ref):
        ns = _norm(d0[...], d1[...], 0)
        o_ref[...] = jnp.dot(x_ref[...] * ns, w_ref[...],
                             preferred_element_type=jnp.float32)

    return pl.pallas_call(
        body,
        grid=(gm, 2),
        in_specs=[
            pl.BlockSpec((_BM, 128), lambda i, j: (i, 0)),
            pl.BlockSpec((128, 128), lambda i, j: (0, j)),
            pl.BlockSpec((_BM, 128), lambda i, j: (i, 0)),
            pl.BlockSpec((_BM, 128), lambda i, j: (gm + i, 0)),
        ],
        out_specs=pl.BlockSpec((_BM, 128), lambda i, j: (j * gm + i, 0)),
        out_shape=jax.ShapeDtypeStruct((2 * n, 128), jnp.float32),
    )(feat, w1, dega, dega)


def _mm2_call(agg1, w2, b1r, dega, n):
    gm = n // _BM

    def body(a_ref, w_ref, b_ref, d0, d1, o_ref):
        k = pl.program_id(1)
        nd = _norm(d0[...], d1[...], 1)
        ns = _norm(d0[...], d1[...], 0)
        x = jnp.maximum(a_ref[...] * nd + b_ref[0], 0.0) * ns
        p = jnp.dot(x, w_ref[...], preferred_element_type=jnp.float32)

        @pl.when(k == 0)
        def _():
            o_ref[...] = p

        @pl.when(k > 0)
        def _():
            o_ref[...] += p

    return pl.pallas_call(
        body,
        grid=(gm, 2),
        in_specs=[
            pl.BlockSpec((_BM, 128), lambda i, k: (k * gm + i, 0)),
            pl.BlockSpec((128, 128), lambda i, k: (k, 0)),
            pl.BlockSpec((1, 1, 128), lambda i, k: (k, 0, 0)),
            pl.BlockSpec((_BM, 128), lambda i, k: (i, 0)),
            pl.BlockSpec((_BM, 128), lambda i, k: (gm + i, 0)),
        ],
        out_specs=pl.BlockSpec((_BM, 128), lambda i, k: (i, 0)),
        out_shape=jax.ShapeDtypeStruct((n, 128), jnp.float32),
    )(agg1, w2, b1r, dega, dega)


def _fin_call(agg2, b2r, dega, n):
    gm = n // _BM

    def body(a0_ref, a1_ref, d0, d1, b_ref, o_ref):
        nd = _norm(d0[...], d1[...], 1)
        o_ref[...] = jnp.maximum(
            (a0_ref[...] + a1_ref[...]) * nd + b_ref[...], 0.0)

    return pl.pallas_call(
        body,
        grid=(gm,),
        in_specs=[
            pl.BlockSpec((_BM, 128), lambda i: (i, 0)),
            pl.BlockSpec((_BM, 128), lambda i: (gm + i, 0)),
            pl.BlockSpec((_BM, 128), lambda i: (i, 0)),
            pl.BlockSpec((_BM, 128), lambda i: (gm + i, 0)),
            pl.BlockSpec((1, 128), lambda i: (0, 0)),
        ],
        out_specs=pl.BlockSpec((_BM, 128), lambda i: (i, 0)),
        out_shape=jax.ShapeDtypeStruct((n, 128), jnp.float32),
    )(agg2, agg2, dega, dega, b2r)


def kernel(feat, edge_index, W1, b1, W2, b2):
    n, in_dim = feat.shape
    e = edge_index.shape[1]
    hid = W1.shape[1]
    out_dim = W2.shape[1]
    assert (in_dim, hid, out_dim) == (128, 256, 128)
    assert e % (32 * _K) == 0 and n % _BM == 0 and n % (2 * _RCH) == 0

    src = edge_index[0]
    dst = edge_index[1]
    z128 = jnp.zeros((_RCH, 128), jnp.float32)
    o2 = jnp.zeros((2 * _K, 128), jnp.float32)
    o2 = o2.at[:_K, 0].set(1.0).at[_K:, 1].set(1.0)

    dega = _deg_call(src, dst, o2, z128, n)
    h1 = _mm1_call(feat, W1, dega, n)
    agg1 = _edge_call(src, dst, h1[:n], h1[n:], z128, n, True)
    h2 = _mm2_call(agg1, W2, b1.reshape(2, 1, 128), dega, n)
    agg2 = _edge_call(src, dst, h2, h2, z128, n, False)
    return _fin_call(agg2, b2.reshape(1, 128), dega, n)
